# baseline (speedup 1.0000x reference)
import functools
import math

import jax
import jax.numpy as jnp
from jax.experimental import pallas as pl
from jax.experimental.pallas import tpu as pltpu

_N = 4096
_D = 64
_ALPHA = 3.0
_K = 12
_BLOCK = 512
_INV_SQRT2 = 1.0 / math.sqrt(2.0)
_GW = 1024


def _fused_kernel(
    emb1_ref, emb2_ref, w1_ref, b1_ref, w2_ref, b2_ref, out_ref, nv2_ref
):
    nv1 = jnp.tanh(
        _ALPHA
        * (
            jnp.dot(emb1_ref[...], w1_ref[...], preferred_element_type=jnp.float32)
            + b1_ref[...]
        )
    )

    @pl.when(pl.program_id(0) == 0)
    def _():
        nv2_ref[...] = jnp.tanh(
            _ALPHA
            * (
                jnp.dot(emb2_ref[...], w2_ref[...], preferred_element_type=jnp.float32)
                + b2_ref[...]
            )
        )

    a = jax.lax.dot_general(
        nv1,
        nv2_ref[...],
        dimension_numbers=(((1,), (1,)), ((), ())),
        preferred_element_type=jnp.float32,
    )
    t = jnp.tanh(_ALPHA * a)
    adj = 0.5 * t * (1.0 + jax.lax.erf(t * _INV_SQRT2))

    rows = adj.shape[0]
    big_neg = jnp.float32(-1e30)
    n_groups = adj.shape[1] // _GW

    g_of_col = jax.lax.broadcasted_iota(jnp.int32, (adj.shape[1], n_groups), 0) // _GW
    g_id = jax.lax.broadcasted_iota(jnp.int32, (adj.shape[1], n_groups), 1)
    gmat = jnp.where(g_of_col == g_id, 1.0, 0.0)

    def class_group_counts(m):
        eq_f = jnp.where(adj == m, 1.0, 0.0)
        gc = jax.lax.dot_general(
            eq_f, gmat, (((1,), (0,)), ((), ())), preferred_element_type=jnp.float32
        )
        return gc, jnp.sum(gc, axis=1, keepdims=True)

    def cond(state):
        v, gc, cnt_gt, cum = state
        return jnp.any(cum < _K)

    def body(state):
        v, gc, cnt_gt, cum = state
        active = cum < _K
        below = jnp.where(adj < v, adj, big_neg)
        m = jnp.max(below, axis=1, keepdims=True)
        gc_n, c = class_group_counts(m)
        v = jnp.where(active, m, v)
        gc = jnp.where(active, gc_n, gc)
        cnt_gt = jnp.where(active, cum, cnt_gt)
        cum = jnp.where(active, cum + c, cum)
        return v, gc, cnt_gt, cum

    m0 = jnp.max(adj, axis=1, keepdims=True)
    gc0, c0 = class_group_counts(m0)
    vk, gc, cnt_gt, _ = jax.lax.while_loop(cond, body, (m0, gc0, c0 * 0.0, c0))

    r = _K - cnt_gt
    eqb = adj == vk
    eq_f = jnp.where(eqb, 1.0, 0.0)

    li = jax.lax.broadcasted_iota(jnp.int32, (n_groups, n_groups), 0)
    lj = jax.lax.broadcasted_iota(jnp.int32, (n_groups, n_groups), 1)
    l32 = jnp.where(li < lj, 1.0, 0.0)
    goff = jax.lax.dot_general(
        gc, l32, (((1,), (0,)), ((), ())), preferred_element_type=jnp.float32
    )

    is_star = (goff < r) & (goff + gc >= r)
    g_iota = jax.lax.broadcasted_iota(jnp.int32, goff.shape, 1).astype(jnp.float32)
    gstar = jnp.sum(jnp.where(is_star, g_iota, 0.0), axis=1, keepdims=True)
    goffs = jnp.sum(jnp.where(is_star, goff, 0.0), axis=1, keepdims=True)
    rloc = r - goffs

    star_f = jnp.where(is_star, 1.0, 0.0)
    acc = jnp.zeros((rows, _GW), jnp.float32)
    for g in range(n_groups):
        acc = acc + star_f[:, g : g + 1] * eq_f[:, g * _GW : (g + 1) * _GW]

    pi = jax.lax.broadcasted_iota(jnp.int32, (_GW, _GW), 0)
    pj = jax.lax.broadcasted_iota(jnp.int32, (_GW, _GW), 1)
    l128 = jnp.where(pi <= pj, 1.0, 0.0)
    prefix = jax.lax.dot_general(
        acc, l128, (((1,), (0,)), ((), ())), preferred_element_type=jnp.float32
    )
    liota = jax.lax.broadcasted_iota(jnp.int32, prefix.shape, 1).astype(jnp.float32)
    hit = jnp.where((prefix == rloc) & (acc > 0.0), liota, float(_GW))
    lstar = jnp.min(hit, axis=1, keepdims=True)
    cstar = (gstar * float(_GW) + lstar).astype(jnp.int32)

    cols = jax.lax.broadcasted_iota(jnp.int32, (1, adj.shape[1]), 1)
    keep = (adj > vk) | (eqb & (cols <= cstar))
    out_ref[...] = jnp.where(keep, adj, 0.0)


@jax.jit
def kernel(emb1, emb2, W1, b1, W2, b2):
    b1 = b1.reshape(1, _D)
    b2 = b2.reshape(1, _D)
    grid = (_N // _BLOCK,)
    return pl.pallas_call(
        _fused_kernel,
        grid=grid,
        in_specs=[
            pl.BlockSpec((_BLOCK, _D), lambda i: (i, 0)),
            pl.BlockSpec((_N, _D), lambda i: (0, 0)),
            pl.BlockSpec((_D, _D), lambda i: (0, 0)),
            pl.BlockSpec((1, _D), lambda i: (0, 0)),
            pl.BlockSpec((_D, _D), lambda i: (0, 0)),
            pl.BlockSpec((1, _D), lambda i: (0, 0)),
        ],
        out_specs=pl.BlockSpec((_BLOCK, _N), lambda i: (i, 0)),
        out_shape=jax.ShapeDtypeStruct((_N, _N), jnp.float32),
        scratch_shapes=[pltpu.VMEM((_N, _D), jnp.float32)],
    )(emb1, emb2, W1, b1, W2, b2)


# speedup vs baseline: 1.0300x; 1.0300x over previous
import functools
import math

import jax
import jax.numpy as jnp
from jax.experimental import pallas as pl
from jax.experimental.pallas import tpu as pltpu

_N = 4096
_D = 64
_ALPHA = 3.0
_K = 12
_BLOCK = 512
_INV_SQRT2 = 1.0 / math.sqrt(2.0)
_GW = 512


def _fused_kernel(
    emb1_ref, emb2_ref, w1_ref, b1_ref, w2_ref, b2_ref, out_ref, nv2_ref
):
    nv1 = jnp.tanh(
        _ALPHA
        * (
            jnp.dot(emb1_ref[...], w1_ref[...], preferred_element_type=jnp.float32)
            + b1_ref[...]
        )
    )

    @pl.when(pl.program_id(0) == 0)
    def _():
        nv2_ref[...] = jnp.tanh(
            _ALPHA
            * (
                jnp.dot(emb2_ref[...], w2_ref[...], preferred_element_type=jnp.float32)
                + b2_ref[...]
            )
        )

    a = jax.lax.dot_general(
        nv1,
        nv2_ref[...],
        dimension_numbers=(((1,), (1,)), ((), ())),
        preferred_element_type=jnp.float32,
    )
    t = jnp.tanh(_ALPHA * a)
    adj = 0.5 * t * (1.0 + jax.lax.erf(t * _INV_SQRT2))

    rows = adj.shape[0]
    big_neg = jnp.float32(-1e30)
    n_groups = adj.shape[1] // _GW

    g_of_col = jax.lax.broadcasted_iota(jnp.int32, (adj.shape[1], n_groups), 0) // _GW
    g_id = jax.lax.broadcasted_iota(jnp.int32, (adj.shape[1], n_groups), 1)
    gmat = jnp.where(g_of_col == g_id, 1.0, 0.0)

    def class_group_counts(m):
        eq_f = jnp.where(adj == m, 1.0, 0.0)
        gc = jax.lax.dot_general(
            eq_f, gmat, (((1,), (0,)), ((), ())), preferred_element_type=jnp.float32
        )
        return gc, jnp.sum(gc, axis=1, keepdims=True)

    def cond(state):
        v, gc, cnt_gt, cum = state
        return jnp.any(cum < _K)

    def body(state):
        v, gc, cnt_gt, cum = state
        active = cum < _K
        below = jnp.where(adj < v, adj, big_neg)
        m = jnp.max(below, axis=1, keepdims=True)
        gc_n, c = class_group_counts(m)
        v = jnp.where(active, m, v)
        gc = jnp.where(active, gc_n, gc)
        cnt_gt = jnp.where(active, cum, cnt_gt)
        cum = jnp.where(active, cum + c, cum)
        return v, gc, cnt_gt, cum

    m0 = jnp.max(adj, axis=1, keepdims=True)
    gc0, c0 = class_group_counts(m0)
    vk, gc, cnt_gt, _ = jax.lax.while_loop(cond, body, (m0, gc0, c0 * 0.0, c0))

    r = _K - cnt_gt
    eqb = adj == vk
    eq_f = jnp.where(eqb, 1.0, 0.0)

    li = jax.lax.broadcasted_iota(jnp.int32, (n_groups, n_groups), 0)
    lj = jax.lax.broadcasted_iota(jnp.int32, (n_groups, n_groups), 1)
    l32 = jnp.where(li < lj, 1.0, 0.0)
    goff = jax.lax.dot_general(
        gc, l32, (((1,), (0,)), ((), ())), preferred_element_type=jnp.float32
    )

    is_star = (goff < r) & (goff + gc >= r)
    g_iota = jax.lax.broadcasted_iota(jnp.int32, goff.shape, 1).astype(jnp.float32)
    gstar = jnp.sum(jnp.where(is_star, g_iota, 0.0), axis=1, keepdims=True)
    goffs = jnp.sum(jnp.where(is_star, goff, 0.0), axis=1, keepdims=True)
    rloc = r - goffs

    star_f = jnp.where(is_star, 1.0, 0.0)
    acc = jnp.zeros((rows, _GW), jnp.float32)
    for g in range(n_groups):
        acc = acc + star_f[:, g : g + 1] * eq_f[:, g * _GW : (g + 1) * _GW]

    pi = jax.lax.broadcasted_iota(jnp.int32, (_GW, _GW), 0)
    pj = jax.lax.broadcasted_iota(jnp.int32, (_GW, _GW), 1)
    l128 = jnp.where(pi <= pj, 1.0, 0.0).astype(jnp.bfloat16)
    prefix = jax.lax.dot_general(
        acc.astype(jnp.bfloat16),
        l128,
        (((1,), (0,)), ((), ())),
        preferred_element_type=jnp.float32,
    )
    liota = jax.lax.broadcasted_iota(jnp.int32, prefix.shape, 1).astype(jnp.float32)
    hit = jnp.where((prefix == rloc) & (acc > 0.0), liota, float(_GW))
    lstar = jnp.min(hit, axis=1, keepdims=True)
    cstar = (gstar * float(_GW) + lstar).astype(jnp.int32)

    cols = jax.lax.broadcasted_iota(jnp.int32, (1, adj.shape[1]), 1)
    keep = (adj > vk) | (eqb & (cols <= cstar))
    out_ref[...] = jnp.where(keep, adj, 0.0)


@jax.jit
def kernel(emb1, emb2, W1, b1, W2, b2):
    b1 = b1.reshape(1, _D)
    b2 = b2.reshape(1, _D)
    grid = (_N // _BLOCK,)
    return pl.pallas_call(
        _fused_kernel,
        grid=grid,
        in_specs=[
            pl.BlockSpec((_BLOCK, _D), lambda i: (i, 0)),
            pl.BlockSpec((_N, _D), lambda i: (0, 0)),
            pl.BlockSpec((_D, _D), lambda i: (0, 0)),
            pl.BlockSpec((1, _D), lambda i: (0, 0)),
            pl.BlockSpec((_D, _D), lambda i: (0, 0)),
            pl.BlockSpec((1, _D), lambda i: (0, 0)),
        ],
        out_specs=pl.BlockSpec((_BLOCK, _N), lambda i: (i, 0)),
        out_shape=jax.ShapeDtypeStruct((_N, _N), jnp.float32),
        scratch_shapes=[pltpu.VMEM((_N, _D), jnp.float32)],
    )(emb1, emb2, W1, b1, W2, b2)


# speedup vs baseline: 1.0362x; 1.0060x over previous
import math

import jax
import jax.numpy as jnp
from jax.experimental import pallas as pl
from jax.experimental.pallas import tpu as pltpu

_N = 4096
_D = 64
_ALPHA = 3.0
_K = 12
_BLOCK = 512
_INV_SQRT2 = 1.0 / math.sqrt(2.0)
_GW = 512


def _fused_kernel(
    emb1_ref, emb2_ref, w1_ref, b1_ref, w2_ref, b2_ref, out_ref, nv2_ref
):
    nv1 = jnp.tanh(
        _ALPHA
        * (
            jnp.dot(emb1_ref[...], w1_ref[...], preferred_element_type=jnp.float32)
            + b1_ref[...]
        )
    )

    @pl.when(pl.program_id(0) == 0)
    def _():
        nv2_ref[...] = jnp.tanh(
            _ALPHA
            * (
                jnp.dot(emb2_ref[...], w2_ref[...], preferred_element_type=jnp.float32)
                + b2_ref[...]
            )
        )

    a = jax.lax.dot_general(
        nv1,
        nv2_ref[...],
        dimension_numbers=(((1,), (1,)), ((), ())),
        preferred_element_type=jnp.float32,
    )
    t = jnp.tanh(_ALPHA * a)
    adj = 0.5 * t * (1.0 + jax.lax.erf(t * _INV_SQRT2))

    rows = adj.shape[0]
    big_neg = jnp.float32(-1e30)
    n_groups = adj.shape[1] // _GW

    g_of_col = jax.lax.broadcasted_iota(jnp.int32, (adj.shape[1], n_groups), 0) // _GW
    g_id = jax.lax.broadcasted_iota(jnp.int32, (adj.shape[1], n_groups), 1)
    gmat = jnp.where(g_of_col == g_id, 1.0, 0.0)

    def class_group_counts(m):
        eq_f = jnp.where(adj == m, 1.0, 0.0)
        gc = jax.lax.dot_general(
            eq_f, gmat, (((1,), (0,)), ((), ())), preferred_element_type=jnp.float32
        )
        return gc, jnp.sum(gc, axis=1, keepdims=True)

    def cond(state):
        v, gc, cnt_gt, cum = state
        return jnp.any(cum < _K)

    def body(state):
        v, gc, cnt_gt, cum = state
        active = cum < _K
        below = jnp.where(adj < v, adj, big_neg)
        m = jnp.max(below, axis=1, keepdims=True)
        gc_n, c = class_group_counts(m)
        v = jnp.where(active, m, v)
        gc = jnp.where(active, gc_n, gc)
        cnt_gt = jnp.where(active, cum, cnt_gt)
        cum = jnp.where(active, cum + c, cum)
        return v, gc, cnt_gt, cum

    m0 = jnp.max(adj, axis=1, keepdims=True)
    gc0, c0 = class_group_counts(m0)
    vk, gc, cnt_gt, _ = jax.lax.while_loop(cond, body, (m0, gc0, c0 * 0.0, c0))

    r = _K - cnt_gt
    eqb = adj == vk
    eq_f = jnp.where(eqb, 1.0, 0.0)

    li = jax.lax.broadcasted_iota(jnp.int32, (n_groups, n_groups), 0)
    lj = jax.lax.broadcasted_iota(jnp.int32, (n_groups, n_groups), 1)
    l32 = jnp.where(li < lj, 1.0, 0.0)
    goff = jax.lax.dot_general(
        gc, l32, (((1,), (0,)), ((), ())), preferred_element_type=jnp.float32
    )

    is_star = (goff < r) & (goff + gc >= r)
    g_iota = jax.lax.broadcasted_iota(jnp.int32, goff.shape, 1).astype(jnp.float32)
    gstar = jnp.sum(jnp.where(is_star, g_iota, 0.0), axis=1, keepdims=True)
    goffs = jnp.sum(jnp.where(is_star, goff, 0.0), axis=1, keepdims=True)
    rloc = r - goffs

    star_f = jnp.where(is_star, 1.0, 0.0)
    acc = jnp.zeros((rows, _GW), jnp.float32)
    for g in range(n_groups):
        acc = acc + star_f[:, g : g + 1] * eq_f[:, g * _GW : (g + 1) * _GW]

    pi = jax.lax.broadcasted_iota(jnp.int32, (_GW, _GW), 0)
    pj = jax.lax.broadcasted_iota(jnp.int32, (_GW, _GW), 1)
    l128 = jnp.where(pi <= pj, 1.0, 0.0)
    prefix = jax.lax.dot_general(
        acc, l128, (((1,), (0,)), ((), ())), preferred_element_type=jnp.float32
    )
    liota = jax.lax.broadcasted_iota(jnp.int32, prefix.shape, 1).astype(jnp.float32)
    hit = jnp.where((prefix == rloc) & (acc > 0.0), liota, float(_GW))
    lstar = jnp.min(hit, axis=1, keepdims=True)
    cstar = (gstar * float(_GW) + lstar).astype(jnp.int32)

    cols = jax.lax.broadcasted_iota(jnp.int32, (1, adj.shape[1]), 1)
    keep = (adj > vk) | (eqb & (cols <= cstar))
    out_ref[...] = jnp.where(keep, adj, 0.0)


@jax.jit
def kernel(emb1, emb2, W1, b1, W2, b2):
    b1 = b1.reshape(1, _D)
    b2 = b2.reshape(1, _D)
    grid = (_N // _BLOCK,)
    return pl.pallas_call(
        _fused_kernel,
        grid=grid,
        in_specs=[
            pl.BlockSpec((_BLOCK, _D), lambda i: (i, 0)),
            pl.BlockSpec((_N, _D), lambda i: (0, 0)),
            pl.BlockSpec((_D, _D), lambda i: (0, 0)),
            pl.BlockSpec((1, _D), lambda i: (0, 0)),
            pl.BlockSpec((_D, _D), lambda i: (0, 0)),
            pl.BlockSpec((1, _D), lambda i: (0, 0)),
        ],
        out_specs=pl.BlockSpec((_BLOCK, _N), lambda i: (i, 0)),
        out_shape=jax.ShapeDtypeStruct((_N, _N), jnp.float32),
        scratch_shapes=[pltpu.VMEM((_N, _D), jnp.float32)],
    )(emb1, emb2, W1, b1, W2, b2)
